# Initial kernel scaffold; baseline (speedup 1.0000x reference)
import jax
import jax.numpy as jnp
from jax.experimental import pallas as pl


def kernel(x, gate_W, gate_b, W1, b1, W2, b2):
    raise NotImplementedError("write your pallas kernel here")


# speedup vs baseline: 7.7908x; 7.7908x over previous
import functools

import jax
import jax.numpy as jnp
from jax import lax
from jax.experimental import pallas as pl
from jax.experimental.pallas import tpu as pltpu
from jax.experimental.pallas import tpu_sc as plsc

T = 2048
D = 768
E = 64
DFF = 2048
TM = 128
NT = 96
P = NT * TM
FFC = 1024
NK = DFF // FFC
NW = 32
PAIRS_W = (2 * T) // NW
TOK_W = T // NW


def _routing_body(xf_ref, gw_ref, gb_ref,
                  d0_ref, d1_ref, s0_ref, s1_ref, te_ref, tv_ref):
    xf = xf_ref[...]
    logits = jnp.dot(xf, gw_ref[...], preferred_element_type=jnp.float32)
    logits = logits + gb_ref[...]
    m = jnp.max(logits, axis=1, keepdims=True)
    ex = jnp.exp(logits - m)
    sm = ex / jnp.sum(ex, axis=1, keepdims=True)

    eidx = lax.broadcasted_iota(jnp.int32, (T, E), 1)
    m0 = jnp.max(sm, axis=1, keepdims=True)
    a0 = jnp.min(jnp.where(sm == m0, eidx, E), axis=1, keepdims=True)
    oh0 = (eidx == a0).astype(jnp.float32)
    sm1 = jnp.where(eidx == a0, -1.0, sm)
    m1 = jnp.max(sm1, axis=1, keepdims=True)
    a1 = jnp.min(jnp.where(sm1 == m1, eidx, E), axis=1, keepdims=True)
    oh1 = (eidx == a1).astype(jnp.float32)

    cnt = jnp.sum(oh0, axis=0, keepdims=True) + jnp.sum(oh1, axis=0, keepdims=True)
    tp = jnp.floor((cnt + (TM - 1)) * (1.0 / TM))
    er = lax.broadcasted_iota(jnp.int32, (E, E), 0)
    ec = lax.broadcasted_iota(jnp.int32, (E, E), 1)
    ltmask = (er < ec).astype(jnp.float32)
    tile_start = jnp.dot(tp, ltmask, preferred_element_type=jnp.float32)
    row_start = tile_start * TM

    ir = lax.broadcasted_iota(jnp.int32, (TM, TM), 0)
    ic = lax.broadcasted_iota(jnp.int32, (TM, TM), 1)
    strict_lt = (ic < ir).astype(jnp.float32)
    pref = jnp.zeros((1, E), jnp.float32)
    ranks = []
    for oh in (oh0, oh1):
        rblocks = []
        for i in range(T // TM):
            ohb = oh[i * TM:(i + 1) * TM]
            gram = lax.dot_general(ohb, ohb, (((1,), (1,)), ((), ())),
                                   preferred_element_type=jnp.float32)
            within = jnp.sum(gram * strict_lt, axis=1, keepdims=True)
            carry = jnp.sum(ohb * pref, axis=1, keepdims=True)
            rblocks.append(within + carry)
            pref = pref + jnp.sum(ohb, axis=0, keepdims=True)
        ranks.append(jnp.concatenate(rblocks, axis=0))
    r0, r1 = ranks

    rs0 = jnp.sum(oh0 * row_start, axis=1, keepdims=True)
    rs1 = jnp.sum(oh1 * row_start, axis=1, keepdims=True)
    d0_ref[...] = (rs0 + r0).astype(jnp.int32)
    d1_ref[...] = (rs1 + r1).astype(jnp.int32)
    s0_ref[...] = m0
    s1_ref[...] = m1

    tt = lax.broadcasted_iota(jnp.int32, (NT, E), 0).astype(jnp.float32)
    ind = ((tt >= tile_start) & (tt < tile_start + tp)).astype(jnp.float32)
    ecol = lax.broadcasted_iota(jnp.int32, (NT, E), 1).astype(jnp.float32)
    te = jnp.sum(ind * ecol, axis=1, keepdims=True)
    tv = jnp.sum(ind, axis=1, keepdims=True)
    te_last = jnp.max(ind * ecol)
    te_ref[...] = jnp.where(tv > 0.0, te, te_last).astype(jnp.int32)
    tv_ref[...] = tv.astype(jnp.int32)


def _routing(xf, gate_W, gate_b, interpret=False):
    f32 = jnp.float32
    return pl.pallas_call(
        _routing_body,
        out_shape=(
            jax.ShapeDtypeStruct((T, 1), jnp.int32),
            jax.ShapeDtypeStruct((T, 1), jnp.int32),
            jax.ShapeDtypeStruct((T, 1), f32),
            jax.ShapeDtypeStruct((T, 1), f32),
            jax.ShapeDtypeStruct((NT, 1), jnp.int32),
            jax.ShapeDtypeStruct((NT, 1), jnp.int32),
        ),
        interpret=interpret,
    )(xf, gate_W, gate_b)


def _ffn_body(te_ref, tv_ref, x_ref, w1_ref, b1_ref, w2_ref, b2_ref, y_ref):
    t = pl.program_id(0)
    k = pl.program_id(1)

    @pl.when(tv_ref[t] > 0)
    def _():
        h = jnp.dot(x_ref[...], w1_ref[0], preferred_element_type=jnp.float32)
        h = jnp.maximum(h + b1_ref[0], 0.0)
        part = jnp.dot(h, w2_ref[0], preferred_element_type=jnp.float32)

        @pl.when(k == 0)
        def _():
            y_ref[...] = part + b2_ref[0]

        @pl.when(k > 0)
        def _():
            y_ref[...] = y_ref[...] + part


def _ffn(te, tv, xs, W1, b1, W2, b2, interpret=False):
    grid_spec = pltpu.PrefetchScalarGridSpec(
        num_scalar_prefetch=2,
        grid=(NT, NK),
        in_specs=[
            pl.BlockSpec((TM, D), lambda t, k, te, tv: (jnp.where(tv[t] > 0, t, 0), 0)),
            pl.BlockSpec((1, D, FFC), lambda t, k, te, tv: (te[t], 0, k)),
            pl.BlockSpec((1, 1, FFC), lambda t, k, te, tv: (te[t] * NK + k, 0, 0)),
            pl.BlockSpec((1, FFC, D), lambda t, k, te, tv: (te[t], k, 0)),
            pl.BlockSpec((1, 1, D), lambda t, k, te, tv: (te[t], 0, 0)),
        ],
        out_specs=pl.BlockSpec((TM, D), lambda t, k, te, tv: (t, 0)),
    )
    return pl.pallas_call(
        _ffn_body,
        grid_spec=grid_spec,
        out_shape=jax.ShapeDtypeStruct((P, D), jnp.float32),
        interpret=interpret,
    )(te, tv, xs, W1, b1.reshape(E * NK, 1, FFC), W2, b2.reshape(E, 1, D))


def _dispatch(xf, dest):
    mesh = plsc.VectorSubcoreMesh(core_axis_name="c", subcore_axis_name="s")

    @functools.partial(
        pl.kernel,
        out_type=jax.ShapeDtypeStruct((P, D), jnp.float32),
        mesh=mesh,
        scratch_types=[
            pltpu.VMEM((PAIRS_W,), jnp.int32),
            pltpu.VMEM((PAIRS_W, D), jnp.float32),
            pltpu.SemaphoreType.DMA,
        ],
    )
    def disp(xf_hbm, dest_hbm, xs_hbm, idx_v, rows_v, sem):
        wid = lax.axis_index("s") * 2 + lax.axis_index("c")
        base = wid * PAIRS_W
        pltpu.sync_copy(dest_hbm.at[pl.ds(base, PAIRS_W)], idx_v)
        src = lax.rem(base, T)
        pltpu.sync_copy(xf_hbm.at[pl.ds(src, PAIRS_W)], rows_v)
        pltpu.async_copy(rows_v, xs_hbm.at[idx_v], sem).wait()

    return disp(xf, dest)


def _combine(y, dest, sflat):
    mesh = plsc.VectorSubcoreMesh(core_axis_name="c", subcore_axis_name="s")

    @functools.partial(
        pl.kernel,
        out_type=jax.ShapeDtypeStruct((T, D), jnp.float32),
        mesh=mesh,
        scratch_types=[
            pltpu.VMEM((TOK_W,), jnp.int32),
            pltpu.VMEM((TOK_W,), jnp.int32),
            pltpu.VMEM((TOK_W + 16,), jnp.float32),
            pltpu.VMEM((TOK_W + 16,), jnp.float32),
            pltpu.VMEM((TOK_W, D), jnp.float32),
            pltpu.VMEM((TOK_W, D), jnp.float32),
            pltpu.SemaphoreType.DMA,
        ],
    )
    def comb(y_hbm, dest_hbm, s_hbm, out_hbm,
             d0_v, d1_v, s0_v, s1_v, r0_v, r1_v, sem):
        wid = lax.axis_index("s") * 2 + lax.axis_index("c")
        tb = wid * TOK_W
        pltpu.sync_copy(dest_hbm.at[pl.ds(tb, TOK_W)], d0_v)
        pltpu.sync_copy(dest_hbm.at[pl.ds(T + tb, TOK_W)], d1_v)
        pltpu.sync_copy(s_hbm.at[pl.ds(tb, TOK_W)], s0_v.at[pl.ds(0, TOK_W)])
        pltpu.sync_copy(s_hbm.at[pl.ds(T + tb, TOK_W)], s1_v.at[pl.ds(0, TOK_W)])
        pltpu.async_copy(y_hbm.at[d0_v], r0_v, sem).wait()
        pltpu.async_copy(y_hbm.at[d1_v], r1_v, sem).wait()

        def tok_body(tok, carry):
            s0b = jnp.full((16,), s0_v[pl.ds(tok, 16)][0], jnp.float32)
            s1b = jnp.full((16,), s1_v[pl.ds(tok, 16)][0], jnp.float32)

            def ch(j, c):
                sl = pl.ds(j * 16, 16)
                r0_v[tok, sl] = r0_v[tok, sl] * s0b + r1_v[tok, sl] * s1b
                return c

            return lax.fori_loop(0, D // 16, ch, carry)

        lax.fori_loop(0, TOK_W, tok_body, 0)
        pltpu.sync_copy(r0_v, out_hbm.at[pl.ds(tb, TOK_W)])

    return comb(y, dest, sflat)


def kernel(x, gate_W, gate_b, W1, b1, W2, b2):
    xf = x.reshape(T, D)
    d0, d1, s0, s1, te, tv = _routing(xf, gate_W, gate_b.reshape(1, E))
    dest = jnp.concatenate([d0[:, 0], d1[:, 0]])
    sflat = jnp.concatenate([s0[:, 0], s1[:, 0]])
    xs = _dispatch(xf, dest)
    y = _ffn(te[:, 0], tv[:, 0], xs, W1, b1, W2, b2)
    out = _combine(y, dest, sflat)
    return out.reshape(1, T, D)


# speedup vs baseline: 10.1209x; 1.2991x over previous
import functools

import jax
import jax.numpy as jnp
from jax import lax
from jax.experimental import pallas as pl
from jax.experimental.pallas import tpu as pltpu
from jax.experimental.pallas import tpu_sc as plsc

T = 2048
D = 768
E = 64
DFF = 2048
TM = 128
NT = 96
P = NT * TM
FFC = 1024
NK = DFF // FFC
NW = 32
PAIRS_W = (2 * T) // NW
TOK_W = T // NW


def _routing_body(xf_ref, gw_ref, gb_ref,
                  d0_ref, d1_ref, s0_ref, s1_ref, te_ref, tv_ref):
    xf = xf_ref[...]
    logits = jnp.dot(xf, gw_ref[...], preferred_element_type=jnp.float32)
    logits = logits + gb_ref[...]
    m = jnp.max(logits, axis=1, keepdims=True)
    ex = jnp.exp(logits - m)
    sm = ex / jnp.sum(ex, axis=1, keepdims=True)

    eidx = lax.broadcasted_iota(jnp.int32, (T, E), 1)
    m0 = jnp.max(sm, axis=1, keepdims=True)
    a0 = jnp.min(jnp.where(sm == m0, eidx, E), axis=1, keepdims=True)
    oh0 = (eidx == a0).astype(jnp.float32)
    sm1 = jnp.where(eidx == a0, -1.0, sm)
    m1 = jnp.max(sm1, axis=1, keepdims=True)
    a1 = jnp.min(jnp.where(sm1 == m1, eidx, E), axis=1, keepdims=True)
    oh1 = (eidx == a1).astype(jnp.float32)

    cnt = jnp.sum(oh0, axis=0, keepdims=True) + jnp.sum(oh1, axis=0, keepdims=True)
    tp = jnp.floor((cnt + (TM - 1)) * (1.0 / TM))
    er = lax.broadcasted_iota(jnp.int32, (E, E), 0)
    ec = lax.broadcasted_iota(jnp.int32, (E, E), 1)
    ltmask = (er < ec).astype(jnp.float32)
    tile_start = jnp.dot(tp, ltmask, preferred_element_type=jnp.float32)
    row_start = tile_start * TM

    ir = lax.broadcasted_iota(jnp.int32, (TM, TM), 0)
    ic = lax.broadcasted_iota(jnp.int32, (TM, TM), 1)
    strict_lt = (ic < ir).astype(jnp.float32)
    pref = jnp.zeros((1, E), jnp.float32)
    ranks = []
    for oh in (oh0, oh1):
        rblocks = []
        for i in range(T // TM):
            ohb = oh[i * TM:(i + 1) * TM]
            gram = lax.dot_general(ohb, ohb, (((1,), (1,)), ((), ())),
                                   preferred_element_type=jnp.float32)
            within = jnp.sum(gram * strict_lt, axis=1, keepdims=True)
            carry = jnp.sum(ohb * pref, axis=1, keepdims=True)
            rblocks.append(within + carry)
            pref = pref + jnp.sum(ohb, axis=0, keepdims=True)
        ranks.append(jnp.concatenate(rblocks, axis=0))
    r0, r1 = ranks

    rs0 = jnp.sum(oh0 * row_start, axis=1, keepdims=True)
    rs1 = jnp.sum(oh1 * row_start, axis=1, keepdims=True)
    d0_ref[...] = (rs0 + r0).astype(jnp.int32)
    d1_ref[...] = (rs1 + r1).astype(jnp.int32)
    s0_ref[...] = m0
    s1_ref[...] = m1

    tt = lax.broadcasted_iota(jnp.int32, (NT, E), 0).astype(jnp.float32)
    ind = ((tt >= tile_start) & (tt < tile_start + tp)).astype(jnp.float32)
    ecol = lax.broadcasted_iota(jnp.int32, (NT, E), 1).astype(jnp.float32)
    te = jnp.sum(ind * ecol, axis=1, keepdims=True)
    tv = jnp.sum(ind, axis=1, keepdims=True)
    te_last = jnp.max(ind * ecol)
    te_ref[...] = jnp.where(tv > 0.0, te, te_last).astype(jnp.int32)
    tv_ref[...] = tv.astype(jnp.int32)


def _routing(xf, gate_W, gate_b, interpret=False):
    f32 = jnp.float32
    return pl.pallas_call(
        _routing_body,
        out_shape=(
            jax.ShapeDtypeStruct((T, 1), jnp.int32),
            jax.ShapeDtypeStruct((T, 1), jnp.int32),
            jax.ShapeDtypeStruct((T, 1), f32),
            jax.ShapeDtypeStruct((T, 1), f32),
            jax.ShapeDtypeStruct((NT, 1), jnp.int32),
            jax.ShapeDtypeStruct((NT, 1), jnp.int32),
        ),
        interpret=interpret,
    )(xf, gate_W, gate_b)


def _ffn_body(te_ref, tv_ref, x_ref, w1_ref, b1_ref, w2_ref, b2_ref, y_ref):
    t = pl.program_id(0)
    k = pl.program_id(1)

    @pl.when(tv_ref[t] > 0)
    def _():
        h = jnp.dot(x_ref[...], w1_ref[0], preferred_element_type=jnp.float32)
        h = jnp.maximum(h + b1_ref[0], 0.0)
        part = jnp.dot(h, w2_ref[0], preferred_element_type=jnp.float32)

        @pl.when(k == 0)
        def _():
            y_ref[...] = part + b2_ref[0]

        @pl.when(k > 0)
        def _():
            y_ref[...] = y_ref[...] + part


def _ffn(te, tv, xs, W1, b1, W2, b2, interpret=False):
    grid_spec = pltpu.PrefetchScalarGridSpec(
        num_scalar_prefetch=2,
        grid=(NT, NK),
        in_specs=[
            pl.BlockSpec((TM, D), lambda t, k, te, tv: (jnp.where(tv[t] > 0, t, 0), 0)),
            pl.BlockSpec((1, D, FFC),
                         lambda t, k, te, tv: (te[t], 0, jnp.where(tv[t] > 0, k, NK - 1))),
            pl.BlockSpec((1, 1, FFC),
                         lambda t, k, te, tv: (te[t] * NK + jnp.where(tv[t] > 0, k, NK - 1), 0, 0)),
            pl.BlockSpec((1, FFC, D),
                         lambda t, k, te, tv: (te[t], jnp.where(tv[t] > 0, k, NK - 1), 0)),
            pl.BlockSpec((1, 1, D), lambda t, k, te, tv: (te[t], 0, 0)),
        ],
        out_specs=pl.BlockSpec((TM, D), lambda t, k, te, tv: (t, 0)),
    )
    return pl.pallas_call(
        _ffn_body,
        grid_spec=grid_spec,
        out_shape=jax.ShapeDtypeStruct((P, D), jnp.float32),
        interpret=interpret,
    )(te, tv, xs, W1, b1.reshape(E * NK, 1, FFC), W2, b2.reshape(E, 1, D))


def _dispatch(xf, dest):
    mesh = plsc.VectorSubcoreMesh(core_axis_name="c", subcore_axis_name="s")

    @functools.partial(
        pl.kernel,
        out_type=jax.ShapeDtypeStruct((P, D), jnp.float32),
        mesh=mesh,
        scratch_types=[
            pltpu.VMEM((PAIRS_W,), jnp.int32),
            pltpu.VMEM((PAIRS_W, D), jnp.float32),
            pltpu.SemaphoreType.DMA,
        ],
    )
    def disp(xf_hbm, dest_hbm, xs_hbm, idx_v, rows_v, sem):
        wid = lax.axis_index("s") * 2 + lax.axis_index("c")
        base = wid * PAIRS_W
        pltpu.sync_copy(dest_hbm.at[pl.ds(base, PAIRS_W)], idx_v)
        src = lax.rem(base, T)
        pltpu.sync_copy(xf_hbm.at[pl.ds(src, PAIRS_W)], rows_v)
        pltpu.async_copy(rows_v, xs_hbm.at[idx_v], sem).wait()

    return disp(xf, dest)


def _combine(y, dest, sflat):
    mesh = plsc.VectorSubcoreMesh(core_axis_name="c", subcore_axis_name="s")

    @functools.partial(
        pl.kernel,
        out_type=jax.ShapeDtypeStruct((T, D), jnp.float32),
        mesh=mesh,
        scratch_types=[
            pltpu.VMEM((TOK_W,), jnp.int32),
            pltpu.VMEM((TOK_W,), jnp.int32),
            pltpu.VMEM((TOK_W + 16,), jnp.float32),
            pltpu.VMEM((TOK_W + 16,), jnp.float32),
            pltpu.VMEM((TOK_W, D), jnp.float32),
            pltpu.VMEM((TOK_W, D), jnp.float32),
            pltpu.SemaphoreType.DMA,
        ],
    )
    def comb(y_hbm, dest_hbm, s_hbm, out_hbm,
             d0_v, d1_v, s0_v, s1_v, r0_v, r1_v, sem):
        wid = lax.axis_index("s") * 2 + lax.axis_index("c")
        tb = wid * TOK_W
        pltpu.sync_copy(dest_hbm.at[pl.ds(tb, TOK_W)], d0_v)
        pltpu.sync_copy(dest_hbm.at[pl.ds(T + tb, TOK_W)], d1_v)
        pltpu.sync_copy(s_hbm.at[pl.ds(tb, TOK_W)], s0_v.at[pl.ds(0, TOK_W)])
        pltpu.sync_copy(s_hbm.at[pl.ds(T + tb, TOK_W)], s1_v.at[pl.ds(0, TOK_W)])
        pltpu.async_copy(y_hbm.at[d0_v], r0_v, sem).wait()
        pltpu.async_copy(y_hbm.at[d1_v], r1_v, sem).wait()

        def tok_body(tok, carry):
            s0b = jnp.full((16,), s0_v[pl.ds(tok, 16)][0], jnp.float32)
            s1b = jnp.full((16,), s1_v[pl.ds(tok, 16)][0], jnp.float32)

            def ch(j, c):
                sl = pl.ds(j * 16, 16)
                r0_v[tok, sl] = r0_v[tok, sl] * s0b + r1_v[tok, sl] * s1b
                return c

            return lax.fori_loop(0, D // 16, ch, carry)

        lax.fori_loop(0, TOK_W, tok_body, 0)
        pltpu.sync_copy(r0_v, out_hbm.at[pl.ds(tb, TOK_W)])

    return comb(y, dest, sflat)


def kernel(x, gate_W, gate_b, W1, b1, W2, b2):
    xf = x.reshape(T, D)
    d0, d1, s0, s1, te, tv = _routing(xf, gate_W, gate_b.reshape(1, E))
    dest = jnp.concatenate([d0[:, 0], d1[:, 0]])
    sflat = jnp.concatenate([s0[:, 0], s1[:, 0]])
    xs = _dispatch(xf, dest)
    y = _ffn(te[:, 0], tv[:, 0], xs, W1, b1, W2, b2)
    out = _combine(y, dest, sflat)
    return out.reshape(1, T, D)


# speedup vs baseline: 10.1692x; 1.0048x over previous
import functools

import jax
import jax.numpy as jnp
from jax import lax
from jax.experimental import pallas as pl
from jax.experimental.pallas import tpu as pltpu
from jax.experimental.pallas import tpu_sc as plsc

T = 2048
D = 768
E = 64
DFF = 2048
TM = 128
NT = 96
P = NT * TM
FFC = 1024
NK = DFF // FFC
NW = 32
PAIRS_W = (2 * T) // NW
TOK_W = T // NW


def _routing_body(xf_ref, gw_ref, gb_ref,
                  d0_ref, d1_ref, s0_ref, s1_ref, te_ref, tv_ref):
    xf = xf_ref[...]
    logits = jnp.dot(xf, gw_ref[...], preferred_element_type=jnp.float32)
    logits = logits + gb_ref[...]
    m = jnp.max(logits, axis=1, keepdims=True)
    ex = jnp.exp(logits - m)
    sm = ex / jnp.sum(ex, axis=1, keepdims=True)

    eidx = lax.broadcasted_iota(jnp.int32, (T, E), 1)
    m0 = jnp.max(sm, axis=1, keepdims=True)
    a0 = jnp.min(jnp.where(sm == m0, eidx, E), axis=1, keepdims=True)
    oh0 = (eidx == a0).astype(jnp.float32)
    sm1 = jnp.where(eidx == a0, -1.0, sm)
    m1 = jnp.max(sm1, axis=1, keepdims=True)
    a1 = jnp.min(jnp.where(sm1 == m1, eidx, E), axis=1, keepdims=True)
    oh1 = (eidx == a1).astype(jnp.float32)

    cnt = jnp.sum(oh0, axis=0, keepdims=True) + jnp.sum(oh1, axis=0, keepdims=True)
    tp = jnp.floor((cnt + (TM - 1)) * (1.0 / TM))
    er = lax.broadcasted_iota(jnp.int32, (E, E), 0)
    ec = lax.broadcasted_iota(jnp.int32, (E, E), 1)
    ltmask = (er < ec).astype(jnp.float32)
    tile_start = jnp.dot(tp, ltmask, preferred_element_type=jnp.float32)
    row_start = tile_start * TM

    ir = lax.broadcasted_iota(jnp.int32, (TM, TM), 0)
    ic = lax.broadcasted_iota(jnp.int32, (TM, TM), 1)
    strict_lt = (ic < ir).astype(jnp.float32)
    pref = jnp.zeros((1, E), jnp.float32)
    ranks = []
    for oh in (oh0, oh1):
        rblocks = []
        for i in range(T // TM):
            ohb = oh[i * TM:(i + 1) * TM]
            gram = lax.dot_general(ohb, ohb, (((1,), (1,)), ((), ())),
                                   preferred_element_type=jnp.float32)
            within = jnp.sum(gram * strict_lt, axis=1, keepdims=True)
            carry = jnp.sum(ohb * pref, axis=1, keepdims=True)
            rblocks.append(within + carry)
            pref = pref + jnp.sum(ohb, axis=0, keepdims=True)
        ranks.append(jnp.concatenate(rblocks, axis=0))
    r0, r1 = ranks

    rs0 = jnp.sum(oh0 * row_start, axis=1, keepdims=True)
    rs1 = jnp.sum(oh1 * row_start, axis=1, keepdims=True)
    d0_ref[...] = (rs0 + r0).astype(jnp.int32)
    d1_ref[...] = (rs1 + r1).astype(jnp.int32)
    s0_ref[...] = m0
    s1_ref[...] = m1

    tt = lax.broadcasted_iota(jnp.int32, (NT, E), 0).astype(jnp.float32)
    ind = ((tt >= tile_start) & (tt < tile_start + tp)).astype(jnp.float32)
    ecol = lax.broadcasted_iota(jnp.int32, (NT, E), 1).astype(jnp.float32)
    te = jnp.sum(ind * ecol, axis=1, keepdims=True)
    tv = jnp.sum(ind, axis=1, keepdims=True)
    te_last = jnp.max(ind * ecol)
    te_ref[...] = jnp.where(tv > 0.0, te, te_last).astype(jnp.int32)
    tv_ref[...] = tv.astype(jnp.int32)


def _routing(xf, gate_W, gate_b, interpret=False):
    f32 = jnp.float32
    return pl.pallas_call(
        _routing_body,
        out_shape=(
            jax.ShapeDtypeStruct((T, 1), jnp.int32),
            jax.ShapeDtypeStruct((T, 1), jnp.int32),
            jax.ShapeDtypeStruct((T, 1), f32),
            jax.ShapeDtypeStruct((T, 1), f32),
            jax.ShapeDtypeStruct((NT, 1), jnp.int32),
            jax.ShapeDtypeStruct((NT, 1), jnp.int32),
        ),
        interpret=interpret,
    )(xf, gate_W, gate_b)


def _ffn_body(te_ref, tv_ref, x_ref, w1_ref, b1_ref, w2_ref, b2_ref, y_ref):
    t = pl.program_id(0)
    k = pl.program_id(1)

    @pl.when(tv_ref[t] > 0)
    def _():
        xb = x_ref[...].astype(jnp.bfloat16)
        h = jnp.dot(xb, w1_ref[0].astype(jnp.bfloat16),
                    preferred_element_type=jnp.float32)
        h = jnp.maximum(h + b1_ref[0], 0.0)
        part = jnp.dot(h.astype(jnp.bfloat16), w2_ref[0].astype(jnp.bfloat16),
                       preferred_element_type=jnp.float32)

        @pl.when(k == 0)
        def _():
            y_ref[...] = part + b2_ref[0]

        @pl.when(k > 0)
        def _():
            y_ref[...] = y_ref[...] + part


def _ffn(te, tv, xs, W1, b1, W2, b2, interpret=False):
    grid_spec = pltpu.PrefetchScalarGridSpec(
        num_scalar_prefetch=2,
        grid=(NT, NK),
        in_specs=[
            pl.BlockSpec((TM, D), lambda t, k, te, tv: (jnp.where(tv[t] > 0, t, 0), 0)),
            pl.BlockSpec((1, D, FFC),
                         lambda t, k, te, tv: (te[t], 0, jnp.where(tv[t] > 0, k, NK - 1))),
            pl.BlockSpec((1, 1, FFC),
                         lambda t, k, te, tv: (te[t] * NK + jnp.where(tv[t] > 0, k, NK - 1), 0, 0)),
            pl.BlockSpec((1, FFC, D),
                         lambda t, k, te, tv: (te[t], jnp.where(tv[t] > 0, k, NK - 1), 0)),
            pl.BlockSpec((1, 1, D), lambda t, k, te, tv: (te[t], 0, 0)),
        ],
        out_specs=pl.BlockSpec((TM, D), lambda t, k, te, tv: (t, 0)),
    )
    return pl.pallas_call(
        _ffn_body,
        grid_spec=grid_spec,
        out_shape=jax.ShapeDtypeStruct((P, D), jnp.float32),
        interpret=interpret,
    )(te, tv, xs, W1, b1.reshape(E * NK, 1, FFC), W2, b2.reshape(E, 1, D))


def _dispatch(xf, dest):
    mesh = plsc.VectorSubcoreMesh(core_axis_name="c", subcore_axis_name="s")

    @functools.partial(
        pl.kernel,
        out_type=jax.ShapeDtypeStruct((P, D), jnp.float32),
        mesh=mesh,
        scratch_types=[
            pltpu.VMEM((PAIRS_W,), jnp.int32),
            pltpu.VMEM((PAIRS_W, D), jnp.float32),
            pltpu.SemaphoreType.DMA,
        ],
    )
    def disp(xf_hbm, dest_hbm, xs_hbm, idx_v, rows_v, sem):
        wid = lax.axis_index("s") * 2 + lax.axis_index("c")
        base = wid * PAIRS_W
        pltpu.sync_copy(dest_hbm.at[pl.ds(base, PAIRS_W)], idx_v)
        src = lax.rem(base, T)
        pltpu.sync_copy(xf_hbm.at[pl.ds(src, PAIRS_W)], rows_v)
        pltpu.async_copy(rows_v, xs_hbm.at[idx_v], sem).wait()

    return disp(xf, dest)


def _combine(y, dest, sflat):
    mesh = plsc.VectorSubcoreMesh(core_axis_name="c", subcore_axis_name="s")

    @functools.partial(
        pl.kernel,
        out_type=jax.ShapeDtypeStruct((T, D), jnp.float32),
        mesh=mesh,
        scratch_types=[
            pltpu.VMEM((TOK_W,), jnp.int32),
            pltpu.VMEM((TOK_W,), jnp.int32),
            pltpu.VMEM((TOK_W + 16,), jnp.float32),
            pltpu.VMEM((TOK_W + 16,), jnp.float32),
            pltpu.VMEM((TOK_W, D), jnp.float32),
            pltpu.VMEM((TOK_W, D), jnp.float32),
            pltpu.SemaphoreType.DMA,
        ],
    )
    def comb(y_hbm, dest_hbm, s_hbm, out_hbm,
             d0_v, d1_v, s0_v, s1_v, r0_v, r1_v, sem):
        wid = lax.axis_index("s") * 2 + lax.axis_index("c")
        tb = wid * TOK_W
        pltpu.sync_copy(dest_hbm.at[pl.ds(tb, TOK_W)], d0_v)
        pltpu.sync_copy(dest_hbm.at[pl.ds(T + tb, TOK_W)], d1_v)
        pltpu.sync_copy(s_hbm.at[pl.ds(tb, TOK_W)], s0_v.at[pl.ds(0, TOK_W)])
        pltpu.sync_copy(s_hbm.at[pl.ds(T + tb, TOK_W)], s1_v.at[pl.ds(0, TOK_W)])
        pltpu.async_copy(y_hbm.at[d0_v], r0_v, sem).wait()
        pltpu.async_copy(y_hbm.at[d1_v], r1_v, sem).wait()

        def tok_body(tok, carry):
            s0b = jnp.full((16,), s0_v[pl.ds(tok, 16)][0], jnp.float32)
            s1b = jnp.full((16,), s1_v[pl.ds(tok, 16)][0], jnp.float32)

            def ch(j, c):
                sl = pl.ds(j * 16, 16)
                r0_v[tok, sl] = r0_v[tok, sl] * s0b + r1_v[tok, sl] * s1b
                return c

            return lax.fori_loop(0, D // 16, ch, carry)

        lax.fori_loop(0, TOK_W, tok_body, 0)
        pltpu.sync_copy(r0_v, out_hbm.at[pl.ds(tb, TOK_W)])

    return comb(y, dest, sflat)


def kernel(x, gate_W, gate_b, W1, b1, W2, b2):
    xf = x.reshape(T, D)
    d0, d1, s0, s1, te, tv = _routing(xf, gate_W, gate_b.reshape(1, E))
    dest = jnp.concatenate([d0[:, 0], d1[:, 0]])
    sflat = jnp.concatenate([s0[:, 0], s1[:, 0]])
    xs = _dispatch(xf, dest)
    y = _ffn(te[:, 0], tv[:, 0], xs, W1, b1, W2, b2)
    out = _combine(y, dest, sflat)
    return out.reshape(1, T, D)


# speedup vs baseline: 10.5518x; 1.0376x over previous
import functools

import jax
import jax.numpy as jnp
from jax import lax
from jax.experimental import pallas as pl
from jax.experimental.pallas import tpu as pltpu
from jax.experimental.pallas import tpu_sc as plsc

T = 2048
D = 768
E = 64
DFF = 2048
TM = 128
NT = 96
P = NT * TM
FFC = 1024
NK = DFF // FFC
NW = 32
PAIRS_W = (2 * T) // NW
TOK_W = T // NW


def _routing_body(xf_ref, gw_ref, gb_ref,
                  d0_ref, d1_ref, s0_ref, s1_ref, te_ref, tv_ref):
    xf = xf_ref[...]
    logits = jnp.dot(xf, gw_ref[...], preferred_element_type=jnp.float32)
    logits = logits + gb_ref[...]
    m = jnp.max(logits, axis=1, keepdims=True)
    ex = jnp.exp(logits - m)
    sm = ex / jnp.sum(ex, axis=1, keepdims=True)

    eidx = lax.broadcasted_iota(jnp.int32, (T, E), 1)
    m0 = jnp.max(sm, axis=1, keepdims=True)
    a0 = jnp.min(jnp.where(sm == m0, eidx, E), axis=1, keepdims=True)
    oh0 = (eidx == a0).astype(jnp.float32)
    sm1 = jnp.where(eidx == a0, -1.0, sm)
    m1 = jnp.max(sm1, axis=1, keepdims=True)
    a1 = jnp.min(jnp.where(sm1 == m1, eidx, E), axis=1, keepdims=True)
    oh1 = (eidx == a1).astype(jnp.float32)

    cnt = jnp.sum(oh0, axis=0, keepdims=True) + jnp.sum(oh1, axis=0, keepdims=True)
    tp = jnp.floor((cnt + (TM - 1)) * (1.0 / TM))
    er = lax.broadcasted_iota(jnp.int32, (E, E), 0)
    ec = lax.broadcasted_iota(jnp.int32, (E, E), 1)
    ltmask = (er < ec).astype(jnp.float32)
    tile_start = jnp.dot(tp, ltmask, preferred_element_type=jnp.float32)
    row_start = tile_start * TM

    ir = lax.broadcasted_iota(jnp.int32, (TM, TM), 0)
    ic = lax.broadcasted_iota(jnp.int32, (TM, TM), 1)
    strict_lt = (ic < ir).astype(jnp.float32)
    pref = jnp.zeros((1, E), jnp.float32)
    ranks = []
    for oh in (oh0, oh1):
        rblocks = []
        for i in range(T // TM):
            ohb = oh[i * TM:(i + 1) * TM]
            gram = lax.dot_general(ohb, ohb, (((1,), (1,)), ((), ())),
                                   preferred_element_type=jnp.float32)
            within = jnp.sum(gram * strict_lt, axis=1, keepdims=True)
            carry = jnp.sum(ohb * pref, axis=1, keepdims=True)
            rblocks.append(within + carry)
            pref = pref + jnp.sum(ohb, axis=0, keepdims=True)
        ranks.append(jnp.concatenate(rblocks, axis=0))
    r0, r1 = ranks

    rs0 = jnp.sum(oh0 * row_start, axis=1, keepdims=True)
    rs1 = jnp.sum(oh1 * row_start, axis=1, keepdims=True)
    d0_ref[...] = (rs0 + r0).astype(jnp.int32)
    d1_ref[...] = (rs1 + r1).astype(jnp.int32)
    s0_ref[...] = m0
    s1_ref[...] = m1

    tt = lax.broadcasted_iota(jnp.int32, (NT, E), 0).astype(jnp.float32)
    ind = ((tt >= tile_start) & (tt < tile_start + tp)).astype(jnp.float32)
    ecol = lax.broadcasted_iota(jnp.int32, (NT, E), 1).astype(jnp.float32)
    te = jnp.sum(ind * ecol, axis=1, keepdims=True)
    tv = jnp.sum(ind, axis=1, keepdims=True)
    te_last = jnp.max(ind * ecol)
    te_ref[...] = jnp.where(tv > 0.0, te, te_last).astype(jnp.int32)
    tv_ref[...] = tv.astype(jnp.int32)


def _routing(xf, gate_W, gate_b, interpret=False):
    f32 = jnp.float32
    return pl.pallas_call(
        _routing_body,
        out_shape=(
            jax.ShapeDtypeStruct((T, 1), jnp.int32),
            jax.ShapeDtypeStruct((T, 1), jnp.int32),
            jax.ShapeDtypeStruct((T, 1), f32),
            jax.ShapeDtypeStruct((T, 1), f32),
            jax.ShapeDtypeStruct((NT, 1), jnp.int32),
            jax.ShapeDtypeStruct((NT, 1), jnp.int32),
        ),
        interpret=interpret,
    )(xf, gate_W, gate_b)


def _ffn_body(te_ref, tv_ref, x_ref, w1_ref, b1_ref, w2_ref, b2_ref, y_ref):
    t = pl.program_id(0)
    k = pl.program_id(1)

    @pl.when(tv_ref[t] > 0)
    def _():
        xb = x_ref[...].astype(jnp.bfloat16)
        h = jnp.dot(xb, w1_ref[0].astype(jnp.bfloat16),
                    preferred_element_type=jnp.float32)
        h = jnp.maximum(h + b1_ref[0], 0.0)
        part = jnp.dot(h.astype(jnp.bfloat16), w2_ref[0].astype(jnp.bfloat16),
                       preferred_element_type=jnp.float32)

        @pl.when(k == 0)
        def _():
            y_ref[...] = part + b2_ref[0]

        @pl.when(k > 0)
        def _():
            y_ref[...] = y_ref[...] + part


def _ffn(te, tv, xs, W1, b1, W2, b2, interpret=False):
    grid_spec = pltpu.PrefetchScalarGridSpec(
        num_scalar_prefetch=2,
        grid=(NT, NK),
        in_specs=[
            pl.BlockSpec((TM, D), lambda t, k, te, tv: (jnp.where(tv[t] > 0, t, 0), 0)),
            pl.BlockSpec((1, D, FFC),
                         lambda t, k, te, tv: (te[t], 0, jnp.where(tv[t] > 0, k, NK - 1))),
            pl.BlockSpec((1, 1, FFC),
                         lambda t, k, te, tv: (te[t] * NK + jnp.where(tv[t] > 0, k, NK - 1), 0, 0)),
            pl.BlockSpec((1, FFC, D),
                         lambda t, k, te, tv: (te[t], jnp.where(tv[t] > 0, k, NK - 1), 0)),
            pl.BlockSpec((1, 1, D), lambda t, k, te, tv: (te[t], 0, 0)),
        ],
        out_specs=pl.BlockSpec((TM, D), lambda t, k, te, tv: (t, 0)),
    )
    return pl.pallas_call(
        _ffn_body,
        grid_spec=grid_spec,
        out_shape=jax.ShapeDtypeStruct((P, D), jnp.float32),
        interpret=interpret,
    )(te, tv, xs, W1, b1.reshape(E * NK, 1, FFC), W2, b2.reshape(E, 1, D))


def _dispatch(xf, dest):
    mesh = plsc.VectorSubcoreMesh(core_axis_name="c", subcore_axis_name="s")

    @functools.partial(
        pl.kernel,
        out_type=jax.ShapeDtypeStruct((P, D), jnp.float32),
        mesh=mesh,
        scratch_types=[
            pltpu.VMEM((PAIRS_W,), jnp.int32),
            pltpu.VMEM((PAIRS_W, D), jnp.float32),
            pltpu.SemaphoreType.DMA,
        ],
    )
    def disp(xf_hbm, dest_hbm, xs_hbm, idx_v, rows_v, sem):
        wid = lax.axis_index("s") * 2 + lax.axis_index("c")
        base = wid * PAIRS_W
        pltpu.sync_copy(dest_hbm.at[pl.ds(base, PAIRS_W)], idx_v)
        src = lax.rem(base, T)
        pltpu.sync_copy(xf_hbm.at[pl.ds(src, PAIRS_W)], rows_v)
        pltpu.async_copy(rows_v, xs_hbm.at[idx_v], sem).wait()

    return disp(xf, dest)


def _combine(y, dest, sflat):
    mesh = plsc.VectorSubcoreMesh(core_axis_name="c", subcore_axis_name="s")

    @functools.partial(
        pl.kernel,
        out_type=jax.ShapeDtypeStruct((T, D), jnp.float32),
        mesh=mesh,
        scratch_types=[
            pltpu.VMEM((TOK_W,), jnp.int32),
            pltpu.VMEM((TOK_W,), jnp.int32),
            pltpu.VMEM((TOK_W + 16,), jnp.float32),
            pltpu.VMEM((TOK_W + 16,), jnp.float32),
            pltpu.VMEM((TOK_W, D), jnp.float32),
            pltpu.VMEM((TOK_W, D), jnp.float32),
            pltpu.SemaphoreType.DMA,
        ],
    )
    def comb(y_hbm, dest_hbm, s_hbm, out_hbm,
             d0_v, d1_v, s0_v, s1_v, r0_v, r1_v, sem):
        wid = lax.axis_index("s") * 2 + lax.axis_index("c")
        tb = wid * TOK_W
        pltpu.sync_copy(dest_hbm.at[pl.ds(tb, TOK_W)], d0_v)
        pltpu.sync_copy(dest_hbm.at[pl.ds(T + tb, TOK_W)], d1_v)
        pltpu.sync_copy(s_hbm.at[pl.ds(tb, TOK_W)], s0_v.at[pl.ds(0, TOK_W)])
        pltpu.sync_copy(s_hbm.at[pl.ds(T + tb, TOK_W)], s1_v.at[pl.ds(0, TOK_W)])
        pltpu.async_copy(y_hbm.at[d0_v], r0_v, sem).wait()
        pltpu.async_copy(y_hbm.at[d1_v], r1_v, sem).wait()

        def tok_body(tok, carry):
            s0b = jnp.full((16,), s0_v[pl.ds(tok, 16)][0], jnp.float32)
            s1b = jnp.full((16,), s1_v[pl.ds(tok, 16)][0], jnp.float32)
            for j in range(D // 16):
                sl = pl.ds(j * 16, 16)
                r0_v[tok, sl] = r0_v[tok, sl] * s0b + r1_v[tok, sl] * s1b
            return carry

        lax.fori_loop(0, TOK_W, tok_body, 0)
        pltpu.sync_copy(r0_v, out_hbm.at[pl.ds(tb, TOK_W)])

    return comb(y, dest, sflat)


def kernel(x, gate_W, gate_b, W1, b1, W2, b2):
    xf = x.reshape(T, D)
    d0, d1, s0, s1, te, tv = _routing(xf, gate_W, gate_b.reshape(1, E))
    dest = jnp.concatenate([d0[:, 0], d1[:, 0]])
    sflat = jnp.concatenate([s0[:, 0], s1[:, 0]])
    xs = _dispatch(xf, dest)
    y = _ffn(te[:, 0], tv[:, 0], xs, W1, b1, W2, b2)
    out = _combine(y, dest, sflat)
    return out.reshape(1, T, D)


# speedup vs baseline: 11.0009x; 1.0426x over previous
import functools

import jax
import jax.numpy as jnp
from jax import lax
from jax.experimental import pallas as pl
from jax.experimental.pallas import tpu as pltpu
from jax.experimental.pallas import tpu_sc as plsc

T = 2048
D = 768
E = 64
DFF = 2048
TM = 128
NT = 96
P = NT * TM
FFC = 2048
NK = DFF // FFC
NW = 32
PAIRS_W = (2 * T) // NW
TOK_W = T // NW


def _routing_body(xf_ref, gw_ref, gb_ref,
                  d0_ref, d1_ref, s0_ref, s1_ref, te_ref, tv_ref):
    xf = xf_ref[...]
    logits = jnp.dot(xf, gw_ref[...], preferred_element_type=jnp.float32)
    logits = logits + gb_ref[...]
    m = jnp.max(logits, axis=1, keepdims=True)
    ex = jnp.exp(logits - m)
    sm = ex / jnp.sum(ex, axis=1, keepdims=True)

    eidx = lax.broadcasted_iota(jnp.int32, (T, E), 1)
    m0 = jnp.max(sm, axis=1, keepdims=True)
    a0 = jnp.min(jnp.where(sm == m0, eidx, E), axis=1, keepdims=True)
    oh0 = (eidx == a0).astype(jnp.float32)
    sm1 = jnp.where(eidx == a0, -1.0, sm)
    m1 = jnp.max(sm1, axis=1, keepdims=True)
    a1 = jnp.min(jnp.where(sm1 == m1, eidx, E), axis=1, keepdims=True)
    oh1 = (eidx == a1).astype(jnp.float32)

    cnt = jnp.sum(oh0, axis=0, keepdims=True) + jnp.sum(oh1, axis=0, keepdims=True)
    tp = jnp.floor((cnt + (TM - 1)) * (1.0 / TM))
    er = lax.broadcasted_iota(jnp.int32, (E, E), 0)
    ec = lax.broadcasted_iota(jnp.int32, (E, E), 1)
    ltmask = (er < ec).astype(jnp.float32)
    tile_start = jnp.dot(tp, ltmask, preferred_element_type=jnp.float32)
    row_start = tile_start * TM

    ir = lax.broadcasted_iota(jnp.int32, (TM, TM), 0)
    ic = lax.broadcasted_iota(jnp.int32, (TM, TM), 1)
    strict_lt = (ic < ir).astype(jnp.float32)
    pref = jnp.zeros((1, E), jnp.float32)
    ranks = []
    for oh in (oh0, oh1):
        rblocks = []
        for i in range(T // TM):
            ohb = oh[i * TM:(i + 1) * TM]
            gram = lax.dot_general(ohb, ohb, (((1,), (1,)), ((), ())),
                                   preferred_element_type=jnp.float32)
            within = jnp.sum(gram * strict_lt, axis=1, keepdims=True)
            carry = jnp.sum(ohb * pref, axis=1, keepdims=True)
            rblocks.append(within + carry)
            pref = pref + jnp.sum(ohb, axis=0, keepdims=True)
        ranks.append(jnp.concatenate(rblocks, axis=0))
    r0, r1 = ranks

    rs0 = jnp.sum(oh0 * row_start, axis=1, keepdims=True)
    rs1 = jnp.sum(oh1 * row_start, axis=1, keepdims=True)
    d0_ref[...] = (rs0 + r0).astype(jnp.int32)
    d1_ref[...] = (rs1 + r1).astype(jnp.int32)
    s0_ref[...] = m0
    s1_ref[...] = m1

    tt = lax.broadcasted_iota(jnp.int32, (NT, E), 0).astype(jnp.float32)
    ind = ((tt >= tile_start) & (tt < tile_start + tp)).astype(jnp.float32)
    ecol = lax.broadcasted_iota(jnp.int32, (NT, E), 1).astype(jnp.float32)
    te = jnp.sum(ind * ecol, axis=1, keepdims=True)
    tv = jnp.sum(ind, axis=1, keepdims=True)
    te_last = jnp.max(ind * ecol)
    te_ref[...] = jnp.where(tv > 0.0, te, te_last).astype(jnp.int32)
    tv_ref[...] = tv.astype(jnp.int32)


def _routing(xf, gate_W, gate_b, interpret=False):
    f32 = jnp.float32
    return pl.pallas_call(
        _routing_body,
        out_shape=(
            jax.ShapeDtypeStruct((T, 1), jnp.int32),
            jax.ShapeDtypeStruct((T, 1), jnp.int32),
            jax.ShapeDtypeStruct((T, 1), f32),
            jax.ShapeDtypeStruct((T, 1), f32),
            jax.ShapeDtypeStruct((NT, 1), jnp.int32),
            jax.ShapeDtypeStruct((NT, 1), jnp.int32),
        ),
        interpret=interpret,
    )(xf, gate_W, gate_b)


def _ffn_body(te_ref, tv_ref, x_ref, w1_ref, b1_ref, w2_ref, b2_ref, y_ref):
    t = pl.program_id(0)
    k = pl.program_id(1)

    @pl.when(tv_ref[t] > 0)
    def _():
        xb = x_ref[...].astype(jnp.bfloat16)
        h = jnp.dot(xb, w1_ref[0].astype(jnp.bfloat16),
                    preferred_element_type=jnp.float32)
        h = jnp.maximum(h + b1_ref[0], 0.0)
        part = jnp.dot(h.astype(jnp.bfloat16), w2_ref[0].astype(jnp.bfloat16),
                       preferred_element_type=jnp.float32)

        @pl.when(k == 0)
        def _():
            y_ref[...] = part + b2_ref[0]

        @pl.when(k > 0)
        def _():
            y_ref[...] = y_ref[...] + part


def _ffn(te, tv, xs, W1, b1, W2, b2, interpret=False):
    grid_spec = pltpu.PrefetchScalarGridSpec(
        num_scalar_prefetch=2,
        grid=(NT, NK),
        in_specs=[
            pl.BlockSpec((TM, D), lambda t, k, te, tv: (jnp.where(tv[t] > 0, t, 0), 0)),
            pl.BlockSpec((1, D, FFC),
                         lambda t, k, te, tv: (te[t], 0, jnp.where(tv[t] > 0, k, NK - 1))),
            pl.BlockSpec((1, 1, FFC),
                         lambda t, k, te, tv: (te[t] * NK + jnp.where(tv[t] > 0, k, NK - 1), 0, 0)),
            pl.BlockSpec((1, FFC, D),
                         lambda t, k, te, tv: (te[t], jnp.where(tv[t] > 0, k, NK - 1), 0)),
            pl.BlockSpec((1, 1, D), lambda t, k, te, tv: (te[t], 0, 0)),
        ],
        out_specs=pl.BlockSpec((TM, D), lambda t, k, te, tv: (t, 0)),
    )
    return pl.pallas_call(
        _ffn_body,
        grid_spec=grid_spec,
        out_shape=jax.ShapeDtypeStruct((P, D), jnp.float32),
        interpret=interpret,
    )(te, tv, xs, W1, b1.reshape(E * NK, 1, FFC), W2, b2.reshape(E, 1, D))


def _dispatch(xf, dest):
    mesh = plsc.VectorSubcoreMesh(core_axis_name="c", subcore_axis_name="s")

    @functools.partial(
        pl.kernel,
        out_type=jax.ShapeDtypeStruct((P, D), jnp.float32),
        mesh=mesh,
        scratch_types=[
            pltpu.VMEM((PAIRS_W,), jnp.int32),
            pltpu.VMEM((PAIRS_W, D), jnp.float32),
            pltpu.SemaphoreType.DMA,
        ],
    )
    def disp(xf_hbm, dest_hbm, xs_hbm, idx_v, rows_v, sem):
        wid = lax.axis_index("s") * 2 + lax.axis_index("c")
        base = wid * PAIRS_W
        pltpu.sync_copy(dest_hbm.at[pl.ds(base, PAIRS_W)], idx_v)
        src = lax.rem(base, T)
        pltpu.sync_copy(xf_hbm.at[pl.ds(src, PAIRS_W)], rows_v)
        pltpu.async_copy(rows_v, xs_hbm.at[idx_v], sem).wait()

    return disp(xf, dest)


def _combine(y, dest, sflat):
    mesh = plsc.VectorSubcoreMesh(core_axis_name="c", subcore_axis_name="s")

    @functools.partial(
        pl.kernel,
        out_type=jax.ShapeDtypeStruct((T, D), jnp.float32),
        mesh=mesh,
        scratch_types=[
            pltpu.VMEM((TOK_W,), jnp.int32),
            pltpu.VMEM((TOK_W,), jnp.int32),
            pltpu.VMEM((TOK_W + 16,), jnp.float32),
            pltpu.VMEM((TOK_W + 16,), jnp.float32),
            pltpu.VMEM((TOK_W, D), jnp.float32),
            pltpu.VMEM((TOK_W, D), jnp.float32),
            pltpu.SemaphoreType.DMA,
        ],
    )
    def comb(y_hbm, dest_hbm, s_hbm, out_hbm,
             d0_v, d1_v, s0_v, s1_v, r0_v, r1_v, sem):
        wid = lax.axis_index("s") * 2 + lax.axis_index("c")
        tb = wid * TOK_W
        pltpu.sync_copy(dest_hbm.at[pl.ds(tb, TOK_W)], d0_v)
        pltpu.sync_copy(dest_hbm.at[pl.ds(T + tb, TOK_W)], d1_v)
        pltpu.sync_copy(s_hbm.at[pl.ds(tb, TOK_W)], s0_v.at[pl.ds(0, TOK_W)])
        pltpu.sync_copy(s_hbm.at[pl.ds(T + tb, TOK_W)], s1_v.at[pl.ds(0, TOK_W)])
        pltpu.async_copy(y_hbm.at[d0_v], r0_v, sem).wait()
        pltpu.async_copy(y_hbm.at[d1_v], r1_v, sem).wait()

        def tok_body(tok, carry):
            s0b = jnp.full((16,), s0_v[pl.ds(tok, 16)][0], jnp.float32)
            s1b = jnp.full((16,), s1_v[pl.ds(tok, 16)][0], jnp.float32)
            for j in range(D // 16):
                sl = pl.ds(j * 16, 16)
                r0_v[tok, sl] = r0_v[tok, sl] * s0b + r1_v[tok, sl] * s1b
            return carry

        lax.fori_loop(0, TOK_W, tok_body, 0)
        pltpu.sync_copy(r0_v, out_hbm.at[pl.ds(tb, TOK_W)])

    return comb(y, dest, sflat)


def kernel(x, gate_W, gate_b, W1, b1, W2, b2):
    xf = x.reshape(T, D)
    d0, d1, s0, s1, te, tv = _routing(xf, gate_W, gate_b.reshape(1, E))
    dest = jnp.concatenate([d0[:, 0], d1[:, 0]])
    sflat = jnp.concatenate([s0[:, 0], s1[:, 0]])
    xs = _dispatch(xf, dest)
    y = _ffn(te[:, 0], tv[:, 0], xs, W1, b1, W2, b2)
    out = _combine(y, dest, sflat)
    return out.reshape(1, T, D)


# speedup vs baseline: 11.4929x; 1.0447x over previous
import functools

import jax
import jax.numpy as jnp
from jax import lax
from jax.experimental import pallas as pl
from jax.experimental.pallas import tpu as pltpu
from jax.experimental.pallas import tpu_sc as plsc

T = 2048
D = 768
E = 64
DFF = 2048
TM = 128
NT = 96
P = NT * TM
FFC = 2048
NK = DFF // FFC
NW = 32
PAIRS_W = (2 * T) // NW
TOK_W = T // NW


def _routing_body(xf_ref, gw_ref, gb_ref,
                  d0_ref, d1_ref, s0_ref, s1_ref, te_ref, tv_ref):
    xf = xf_ref[...]
    logits = jnp.dot(xf, gw_ref[...], preferred_element_type=jnp.float32)
    logits = logits + gb_ref[...]
    m = jnp.max(logits, axis=1, keepdims=True)
    ex = jnp.exp(logits - m)
    sm = ex / jnp.sum(ex, axis=1, keepdims=True)

    eidx = lax.broadcasted_iota(jnp.int32, (T, E), 1)
    m0 = jnp.max(sm, axis=1, keepdims=True)
    a0 = jnp.min(jnp.where(sm == m0, eidx, E), axis=1, keepdims=True)
    oh0 = (eidx == a0).astype(jnp.float32)
    sm1 = jnp.where(eidx == a0, -1.0, sm)
    m1 = jnp.max(sm1, axis=1, keepdims=True)
    a1 = jnp.min(jnp.where(sm1 == m1, eidx, E), axis=1, keepdims=True)
    oh1 = (eidx == a1).astype(jnp.float32)

    cnt = jnp.sum(oh0, axis=0, keepdims=True) + jnp.sum(oh1, axis=0, keepdims=True)
    tp = jnp.floor((cnt + (TM - 1)) * (1.0 / TM))
    er = lax.broadcasted_iota(jnp.int32, (E, E), 0)
    ec = lax.broadcasted_iota(jnp.int32, (E, E), 1)
    ltmask = (er < ec).astype(jnp.float32)
    tile_start = jnp.dot(tp, ltmask, preferred_element_type=jnp.float32)
    row_start = tile_start * TM

    ir = lax.broadcasted_iota(jnp.int32, (TM, TM), 0)
    ic = lax.broadcasted_iota(jnp.int32, (TM, TM), 1)
    strict_lt = (ic < ir).astype(jnp.float32)
    pref = jnp.zeros((1, E), jnp.float32)
    ranks = []
    for oh in (oh0, oh1):
        rblocks = []
        for i in range(T // TM):
            ohb = oh[i * TM:(i + 1) * TM]
            gram = lax.dot_general(ohb, ohb, (((1,), (1,)), ((), ())),
                                   preferred_element_type=jnp.float32)
            within = jnp.sum(gram * strict_lt, axis=1, keepdims=True)
            carry = jnp.sum(ohb * pref, axis=1, keepdims=True)
            rblocks.append(within + carry)
            pref = pref + jnp.sum(ohb, axis=0, keepdims=True)
        ranks.append(jnp.concatenate(rblocks, axis=0))
    r0, r1 = ranks

    rs0 = jnp.sum(oh0 * row_start, axis=1, keepdims=True)
    rs1 = jnp.sum(oh1 * row_start, axis=1, keepdims=True)
    d0_ref[...] = (rs0 + r0).astype(jnp.int32)
    d1_ref[...] = (rs1 + r1).astype(jnp.int32)
    s0_ref[...] = m0
    s1_ref[...] = m1

    tt = lax.broadcasted_iota(jnp.int32, (NT, E), 0).astype(jnp.float32)
    ind = ((tt >= tile_start) & (tt < tile_start + tp)).astype(jnp.float32)
    ecol = lax.broadcasted_iota(jnp.int32, (NT, E), 1).astype(jnp.float32)
    te = jnp.sum(ind * ecol, axis=1, keepdims=True)
    tv = jnp.sum(ind, axis=1, keepdims=True)
    te_last = jnp.max(ind * ecol)
    te_ref[...] = jnp.where(tv > 0.0, te, te_last).astype(jnp.int32)
    tv_ref[...] = tv.astype(jnp.int32)


def _routing(xf, gate_W, gate_b, interpret=False):
    f32 = jnp.float32
    return pl.pallas_call(
        _routing_body,
        out_shape=(
            jax.ShapeDtypeStruct((T, 1), jnp.int32),
            jax.ShapeDtypeStruct((T, 1), jnp.int32),
            jax.ShapeDtypeStruct((T, 1), f32),
            jax.ShapeDtypeStruct((T, 1), f32),
            jax.ShapeDtypeStruct((NT, 1), jnp.int32),
            jax.ShapeDtypeStruct((NT, 1), jnp.int32),
        ),
        interpret=interpret,
    )(xf, gate_W, gate_b)


def _ffn_body(te_ref, tv_ref, x_ref, w1_ref, b1_ref, w2_ref, b2_ref, y_ref):
    t = pl.program_id(0)
    k = pl.program_id(1)

    @pl.when(tv_ref[t] > 0)
    def _():
        xb = x_ref[...].astype(jnp.bfloat16)
        h = jnp.dot(xb, w1_ref[0].astype(jnp.bfloat16),
                    preferred_element_type=jnp.float32)
        h = jnp.maximum(h + b1_ref[0], 0.0)
        part = jnp.dot(h.astype(jnp.bfloat16), w2_ref[0].astype(jnp.bfloat16),
                       preferred_element_type=jnp.float32)

        @pl.when(k == 0)
        def _():
            y_ref[...] = part + b2_ref[0]

        @pl.when(k > 0)
        def _():
            y_ref[...] = y_ref[...] + part


def _ffn(te, tv, xs, W1, b1, W2, b2, interpret=False):
    grid_spec = pltpu.PrefetchScalarGridSpec(
        num_scalar_prefetch=2,
        grid=(NT, NK),
        in_specs=[
            pl.BlockSpec((TM, D), lambda t, k, te, tv: (jnp.where(tv[t] > 0, t, 0), 0)),
            pl.BlockSpec((1, D, FFC),
                         lambda t, k, te, tv: (te[t], 0, jnp.where(tv[t] > 0, k, NK - 1))),
            pl.BlockSpec((1, 1, FFC),
                         lambda t, k, te, tv: (te[t] * NK + jnp.where(tv[t] > 0, k, NK - 1), 0, 0)),
            pl.BlockSpec((1, FFC, D),
                         lambda t, k, te, tv: (te[t], jnp.where(tv[t] > 0, k, NK - 1), 0)),
            pl.BlockSpec((1, 1, D), lambda t, k, te, tv: (te[t], 0, 0)),
        ],
        out_specs=pl.BlockSpec((TM, D),
                               lambda t, k, te, tv: (jnp.where(tv[t] > 0, t, NT - 1), 0)),
    )
    return pl.pallas_call(
        _ffn_body,
        grid_spec=grid_spec,
        out_shape=jax.ShapeDtypeStruct((P, D), jnp.float32),
        interpret=interpret,
    )(te, tv, xs, W1, b1.reshape(E * NK, 1, FFC), W2, b2.reshape(E, 1, D))


def _dispatch(xf, dest):
    mesh = plsc.VectorSubcoreMesh(core_axis_name="c", subcore_axis_name="s")

    @functools.partial(
        pl.kernel,
        out_type=jax.ShapeDtypeStruct((P, D), jnp.float32),
        mesh=mesh,
        scratch_types=[
            pltpu.VMEM((2, PAIRS_W // 2), jnp.int32),
            pltpu.VMEM((PAIRS_W, D), jnp.float32),
            pltpu.SemaphoreType.DMA,
            pltpu.SemaphoreType.DMA,
            pltpu.SemaphoreType.DMA,
        ],
    )
    def disp(xf_hbm, dest_hbm, xs_hbm, idx_v, rows_v, sem0, sem1, sem2):
        wid = lax.axis_index("s") * 2 + lax.axis_index("c")
        base = wid * PAIRS_W
        half = PAIRS_W // 2
        pltpu.sync_copy(dest_hbm.at[pl.ds(base, half)], idx_v.at[0])
        pltpu.sync_copy(dest_hbm.at[pl.ds(base + half, half)], idx_v.at[1])
        src = lax.rem(base, T)
        c0 = pltpu.async_copy(xf_hbm.at[pl.ds(src, half)],
                              rows_v.at[pl.ds(0, half)], sem0)
        c1 = pltpu.async_copy(xf_hbm.at[pl.ds(src + half, half)],
                              rows_v.at[pl.ds(half, half)], sem1)
        c0.wait()
        s0c = pltpu.async_copy(rows_v.at[pl.ds(0, half)],
                               xs_hbm.at[idx_v.at[0]], sem2)
        c1.wait()
        s1c = pltpu.async_copy(rows_v.at[pl.ds(half, half)],
                               xs_hbm.at[idx_v.at[1]], sem0)
        s0c.wait()
        s1c.wait()

    return disp(xf, dest)


def _combine(y, dest, sflat):
    mesh = plsc.VectorSubcoreMesh(core_axis_name="c", subcore_axis_name="s")

    @functools.partial(
        pl.kernel,
        out_type=jax.ShapeDtypeStruct((T, D), jnp.float32),
        mesh=mesh,
        scratch_types=[
            pltpu.VMEM((TOK_W,), jnp.int32),
            pltpu.VMEM((TOK_W,), jnp.int32),
            pltpu.VMEM((TOK_W + 16,), jnp.float32),
            pltpu.VMEM((TOK_W + 16,), jnp.float32),
            pltpu.VMEM((TOK_W, D), jnp.float32),
            pltpu.VMEM((TOK_W, D), jnp.float32),
            pltpu.SemaphoreType.DMA,
            pltpu.SemaphoreType.DMA,
        ],
    )
    def comb(y_hbm, dest_hbm, s_hbm, out_hbm,
             d0_v, d1_v, s0_v, s1_v, r0_v, r1_v, sem, semb):
        wid = lax.axis_index("s") * 2 + lax.axis_index("c")
        tb = wid * TOK_W
        pltpu.sync_copy(dest_hbm.at[pl.ds(tb, TOK_W)], d0_v)
        pltpu.sync_copy(dest_hbm.at[pl.ds(T + tb, TOK_W)], d1_v)
        pltpu.sync_copy(s_hbm.at[pl.ds(tb, TOK_W)], s0_v.at[pl.ds(0, TOK_W)])
        pltpu.sync_copy(s_hbm.at[pl.ds(T + tb, TOK_W)], s1_v.at[pl.ds(0, TOK_W)])
        g0 = pltpu.async_copy(y_hbm.at[d0_v], r0_v, sem)
        g1 = pltpu.async_copy(y_hbm.at[d1_v], r1_v, semb)
        g0.wait()
        g1.wait()

        def tok_body(tok, carry):
            s0b = jnp.full((16,), s0_v[pl.ds(tok, 16)][0], jnp.float32)
            s1b = jnp.full((16,), s1_v[pl.ds(tok, 16)][0], jnp.float32)
            for j in range(D // 16):
                sl = pl.ds(j * 16, 16)
                r0_v[tok, sl] = r0_v[tok, sl] * s0b + r1_v[tok, sl] * s1b
            return carry

        lax.fori_loop(0, TOK_W, tok_body, 0)
        pltpu.sync_copy(r0_v, out_hbm.at[pl.ds(tb, TOK_W)])

    return comb(y, dest, sflat)


def kernel(x, gate_W, gate_b, W1, b1, W2, b2):
    xf = x.reshape(T, D)
    d0, d1, s0, s1, te, tv = _routing(xf, gate_W, gate_b.reshape(1, E))
    dest = jnp.concatenate([d0[:, 0], d1[:, 0]])
    sflat = jnp.concatenate([s0[:, 0], s1[:, 0]])
    xs = _dispatch(xf, dest)
    y = _ffn(te[:, 0], tv[:, 0], xs, W1, b1, W2, b2)
    out = _combine(y, dest, sflat)
    return out.reshape(1, T, D)


# speedup vs baseline: 11.5102x; 1.0015x over previous
import functools

import jax
import jax.numpy as jnp
from jax import lax
from jax.experimental import pallas as pl
from jax.experimental.pallas import tpu as pltpu
from jax.experimental.pallas import tpu_sc as plsc

T = 2048
D = 768
E = 64
DFF = 2048
TM = 96
NT = 106
P = NT * TM
RB = 128
FFC = 2048
NK = DFF // FFC
NW = 32
PAIRS_W = (2 * T) // NW
TOK_W = T // NW


def _routing_body(xf_ref, gw_ref, gb_ref,
                  d0_ref, d1_ref, s0_ref, s1_ref, te_ref, tv_ref):
    xf = xf_ref[...]
    logits = jnp.dot(xf, gw_ref[...], preferred_element_type=jnp.float32)
    logits = logits + gb_ref[...]
    m = jnp.max(logits, axis=1, keepdims=True)
    ex = jnp.exp(logits - m)
    sm = ex / jnp.sum(ex, axis=1, keepdims=True)

    eidx = lax.broadcasted_iota(jnp.int32, (T, E), 1)
    m0 = jnp.max(sm, axis=1, keepdims=True)
    a0 = jnp.min(jnp.where(sm == m0, eidx, E), axis=1, keepdims=True)
    oh0 = (eidx == a0).astype(jnp.float32)
    sm1 = jnp.where(eidx == a0, -1.0, sm)
    m1 = jnp.max(sm1, axis=1, keepdims=True)
    a1 = jnp.min(jnp.where(sm1 == m1, eidx, E), axis=1, keepdims=True)
    oh1 = (eidx == a1).astype(jnp.float32)

    cnt = jnp.sum(oh0, axis=0, keepdims=True) + jnp.sum(oh1, axis=0, keepdims=True)
    tp = jnp.floor((cnt + (TM - 1)) * (1.0 / TM))
    er = lax.broadcasted_iota(jnp.int32, (E, E), 0)
    ec = lax.broadcasted_iota(jnp.int32, (E, E), 1)
    ltmask = (er < ec).astype(jnp.float32)
    tile_start = jnp.dot(tp, ltmask, preferred_element_type=jnp.float32)
    row_start = tile_start * TM

    ir = lax.broadcasted_iota(jnp.int32, (RB, RB), 0)
    ic = lax.broadcasted_iota(jnp.int32, (RB, RB), 1)
    strict_lt = (ic < ir).astype(jnp.float32)
    pref = jnp.zeros((1, E), jnp.float32)
    ranks = []
    for oh in (oh0, oh1):
        rblocks = []
        for i in range(T // RB):
            ohb = oh[i * RB:(i + 1) * RB]
            gram = lax.dot_general(ohb, ohb, (((1,), (1,)), ((), ())),
                                   preferred_element_type=jnp.float32)
            within = jnp.sum(gram * strict_lt, axis=1, keepdims=True)
            carry = jnp.sum(ohb * pref, axis=1, keepdims=True)
            rblocks.append(within + carry)
            pref = pref + jnp.sum(ohb, axis=0, keepdims=True)
        ranks.append(jnp.concatenate(rblocks, axis=0))
    r0, r1 = ranks

    rs0 = jnp.sum(oh0 * row_start, axis=1, keepdims=True)
    rs1 = jnp.sum(oh1 * row_start, axis=1, keepdims=True)
    d0_ref[...] = (rs0 + r0).astype(jnp.int32)
    d1_ref[...] = (rs1 + r1).astype(jnp.int32)
    s0_ref[...] = m0
    s1_ref[...] = m1

    tt = lax.broadcasted_iota(jnp.int32, (NT, E), 0).astype(jnp.float32)
    ind = ((tt >= tile_start) & (tt < tile_start + tp)).astype(jnp.float32)
    ecol = lax.broadcasted_iota(jnp.int32, (NT, E), 1).astype(jnp.float32)
    te = jnp.sum(ind * ecol, axis=1, keepdims=True)
    tv = jnp.sum(ind, axis=1, keepdims=True)
    te_last = jnp.max(ind * ecol)
    te_ref[...] = jnp.where(tv > 0.0, te, te_last).astype(jnp.int32)
    tv_ref[...] = tv.astype(jnp.int32)


def _routing(xf, gate_W, gate_b, interpret=False):
    f32 = jnp.float32
    return pl.pallas_call(
        _routing_body,
        out_shape=(
            jax.ShapeDtypeStruct((T, 1), jnp.int32),
            jax.ShapeDtypeStruct((T, 1), jnp.int32),
            jax.ShapeDtypeStruct((T, 1), f32),
            jax.ShapeDtypeStruct((T, 1), f32),
            jax.ShapeDtypeStruct((NT, 1), jnp.int32),
            jax.ShapeDtypeStruct((NT, 1), jnp.int32),
        ),
        interpret=interpret,
    )(xf, gate_W, gate_b)


def _ffn_body(te_ref, tv_ref, x_ref, w1_ref, b1_ref, w2_ref, b2_ref, y_ref):
    t = pl.program_id(0)
    k = pl.program_id(1)

    @pl.when(tv_ref[t] > 0)
    def _():
        xb = x_ref[...].astype(jnp.bfloat16)
        h = jnp.dot(xb, w1_ref[0].astype(jnp.bfloat16),
                    preferred_element_type=jnp.float32)
        h = jnp.maximum(h + b1_ref[0], 0.0)
        part = jnp.dot(h.astype(jnp.bfloat16), w2_ref[0].astype(jnp.bfloat16),
                       preferred_element_type=jnp.float32)

        @pl.when(k == 0)
        def _():
            y_ref[...] = part + b2_ref[0]

        @pl.when(k > 0)
        def _():
            y_ref[...] = y_ref[...] + part


def _ffn(te, tv, xs, W1, b1, W2, b2, interpret=False):
    grid_spec = pltpu.PrefetchScalarGridSpec(
        num_scalar_prefetch=2,
        grid=(NT, NK),
        in_specs=[
            pl.BlockSpec((TM, D), lambda t, k, te, tv: (jnp.where(tv[t] > 0, t, 0), 0)),
            pl.BlockSpec((1, D, FFC),
                         lambda t, k, te, tv: (te[t], 0, jnp.where(tv[t] > 0, k, NK - 1))),
            pl.BlockSpec((1, 1, FFC),
                         lambda t, k, te, tv: (te[t] * NK + jnp.where(tv[t] > 0, k, NK - 1), 0, 0)),
            pl.BlockSpec((1, FFC, D),
                         lambda t, k, te, tv: (te[t], jnp.where(tv[t] > 0, k, NK - 1), 0)),
            pl.BlockSpec((1, 1, D), lambda t, k, te, tv: (te[t], 0, 0)),
        ],
        out_specs=pl.BlockSpec((TM, D),
                               lambda t, k, te, tv: (jnp.where(tv[t] > 0, t, NT - 1), 0)),
    )
    return pl.pallas_call(
        _ffn_body,
        grid_spec=grid_spec,
        out_shape=jax.ShapeDtypeStruct((P, D), jnp.float32),
        interpret=interpret,
    )(te, tv, xs, W1, b1.reshape(E * NK, 1, FFC), W2, b2.reshape(E, 1, D))


def _dispatch(xf, dest):
    mesh = plsc.VectorSubcoreMesh(core_axis_name="c", subcore_axis_name="s")

    @functools.partial(
        pl.kernel,
        out_type=jax.ShapeDtypeStruct((P, D), jnp.float32),
        mesh=mesh,
        scratch_types=[
            pltpu.VMEM((2, PAIRS_W // 2), jnp.int32),
            pltpu.VMEM((PAIRS_W, D), jnp.float32),
            pltpu.SemaphoreType.DMA,
            pltpu.SemaphoreType.DMA,
            pltpu.SemaphoreType.DMA,
        ],
    )
    def disp(xf_hbm, dest_hbm, xs_hbm, idx_v, rows_v, sem0, sem1, sem2):
        wid = lax.axis_index("s") * 2 + lax.axis_index("c")
        base = wid * PAIRS_W
        half = PAIRS_W // 2
        pltpu.sync_copy(dest_hbm.at[pl.ds(base, half)], idx_v.at[0])
        pltpu.sync_copy(dest_hbm.at[pl.ds(base + half, half)], idx_v.at[1])
        src = lax.rem(base, T)
        c0 = pltpu.async_copy(xf_hbm.at[pl.ds(src, half)],
                              rows_v.at[pl.ds(0, half)], sem0)
        c1 = pltpu.async_copy(xf_hbm.at[pl.ds(src + half, half)],
                              rows_v.at[pl.ds(half, half)], sem1)
        c0.wait()
        s0c = pltpu.async_copy(rows_v.at[pl.ds(0, half)],
                               xs_hbm.at[idx_v.at[0]], sem2)
        c1.wait()
        s1c = pltpu.async_copy(rows_v.at[pl.ds(half, half)],
                               xs_hbm.at[idx_v.at[1]], sem0)
        s0c.wait()
        s1c.wait()

    return disp(xf, dest)


def _combine(y, dest, sflat):
    mesh = plsc.VectorSubcoreMesh(core_axis_name="c", subcore_axis_name="s")

    @functools.partial(
        pl.kernel,
        out_type=jax.ShapeDtypeStruct((T, D), jnp.float32),
        mesh=mesh,
        scratch_types=[
            pltpu.VMEM((TOK_W,), jnp.int32),
            pltpu.VMEM((TOK_W,), jnp.int32),
            pltpu.VMEM((TOK_W + 16,), jnp.float32),
            pltpu.VMEM((TOK_W + 16,), jnp.float32),
            pltpu.VMEM((TOK_W, D), jnp.float32),
            pltpu.VMEM((TOK_W, D), jnp.float32),
            pltpu.SemaphoreType.DMA,
            pltpu.SemaphoreType.DMA,
        ],
    )
    def comb(y_hbm, dest_hbm, s_hbm, out_hbm,
             d0_v, d1_v, s0_v, s1_v, r0_v, r1_v, sem, semb):
        wid = lax.axis_index("s") * 2 + lax.axis_index("c")
        tb = wid * TOK_W
        pltpu.sync_copy(dest_hbm.at[pl.ds(tb, TOK_W)], d0_v)
        pltpu.sync_copy(dest_hbm.at[pl.ds(T + tb, TOK_W)], d1_v)
        pltpu.sync_copy(s_hbm.at[pl.ds(tb, TOK_W)], s0_v.at[pl.ds(0, TOK_W)])
        pltpu.sync_copy(s_hbm.at[pl.ds(T + tb, TOK_W)], s1_v.at[pl.ds(0, TOK_W)])
        g0 = pltpu.async_copy(y_hbm.at[d0_v], r0_v, sem)
        g1 = pltpu.async_copy(y_hbm.at[d1_v], r1_v, semb)
        g0.wait()
        g1.wait()

        def tok_body(tok, carry):
            s0b = jnp.full((16,), s0_v[pl.ds(tok, 16)][0], jnp.float32)
            s1b = jnp.full((16,), s1_v[pl.ds(tok, 16)][0], jnp.float32)
            for j in range(D // 16):
                sl = pl.ds(j * 16, 16)
                r0_v[tok, sl] = r0_v[tok, sl] * s0b + r1_v[tok, sl] * s1b
            return carry

        lax.fori_loop(0, TOK_W, tok_body, 0)
        pltpu.sync_copy(r0_v, out_hbm.at[pl.ds(tb, TOK_W)])

    return comb(y, dest, sflat)


def kernel(x, gate_W, gate_b, W1, b1, W2, b2):
    xf = x.reshape(T, D)
    d0, d1, s0, s1, te, tv = _routing(xf, gate_W, gate_b.reshape(1, E))
    dest = jnp.concatenate([d0[:, 0], d1[:, 0]])
    sflat = jnp.concatenate([s0[:, 0], s1[:, 0]])
    xs = _dispatch(xf, dest)
    y = _ffn(te[:, 0], tv[:, 0], xs, W1, b1, W2, b2)
    out = _combine(y, dest, sflat)
    return out.reshape(1, T, D)


# speedup vs baseline: 11.5244x; 1.0012x over previous
import functools

import jax
import jax.numpy as jnp
from jax import lax
from jax.experimental import pallas as pl
from jax.experimental.pallas import tpu as pltpu
from jax.experimental.pallas import tpu_sc as plsc

T = 2048
D = 768
E = 64
DFF = 2048
TM = 128
NT = 96
P = NT * TM
RB = 128
FFC = 2048
NK = DFF // FFC
NW = 32
PAIRS_W = (2 * T) // NW
TOK_W = T // NW


def _routing_body(xf_ref, gw_ref, gb_ref,
                  d0_ref, d1_ref, s0_ref, s1_ref, te_ref, tv_ref):
    xf = xf_ref[...]
    logits = jnp.dot(xf, gw_ref[...], preferred_element_type=jnp.float32)
    logits = logits + gb_ref[...]
    m = jnp.max(logits, axis=1, keepdims=True)
    ex = jnp.exp(logits - m)
    sm = ex / jnp.sum(ex, axis=1, keepdims=True)

    eidx = lax.broadcasted_iota(jnp.int32, (T, E), 1)
    m0 = jnp.max(sm, axis=1, keepdims=True)
    a0 = jnp.min(jnp.where(sm == m0, eidx, E), axis=1, keepdims=True)
    oh0 = (eidx == a0).astype(jnp.float32)
    sm1 = jnp.where(eidx == a0, -1.0, sm)
    m1 = jnp.max(sm1, axis=1, keepdims=True)
    a1 = jnp.min(jnp.where(sm1 == m1, eidx, E), axis=1, keepdims=True)
    oh1 = (eidx == a1).astype(jnp.float32)

    cnt = jnp.sum(oh0, axis=0, keepdims=True) + jnp.sum(oh1, axis=0, keepdims=True)
    tp = jnp.floor((cnt + (TM - 1)) * (1.0 / TM))
    er = lax.broadcasted_iota(jnp.int32, (E, E), 0)
    ec = lax.broadcasted_iota(jnp.int32, (E, E), 1)
    ltmask = (er < ec).astype(jnp.float32)
    tile_start = jnp.dot(tp, ltmask, preferred_element_type=jnp.float32)
    row_start = tile_start * TM

    ir = lax.broadcasted_iota(jnp.int32, (RB, RB), 0)
    ic = lax.broadcasted_iota(jnp.int32, (RB, RB), 1)
    strict_lt = (ic < ir).astype(jnp.float32)
    pref = jnp.zeros((1, E), jnp.float32)
    ranks = []
    for oh in (oh0, oh1):
        rblocks = []
        for i in range(T // RB):
            ohb = oh[i * RB:(i + 1) * RB]
            gram = lax.dot_general(ohb, ohb, (((1,), (1,)), ((), ())),
                                   preferred_element_type=jnp.float32)
            within = jnp.sum(gram * strict_lt, axis=1, keepdims=True)
            carry = jnp.sum(ohb * pref, axis=1, keepdims=True)
            rblocks.append(within + carry)
            pref = pref + jnp.sum(ohb, axis=0, keepdims=True)
        ranks.append(jnp.concatenate(rblocks, axis=0))
    r0, r1 = ranks

    rs0 = jnp.sum(oh0 * row_start, axis=1, keepdims=True)
    rs1 = jnp.sum(oh1 * row_start, axis=1, keepdims=True)
    d0_ref[...] = (rs0 + r0).astype(jnp.int32)
    d1_ref[...] = (rs1 + r1).astype(jnp.int32)
    s0_ref[...] = m0
    s1_ref[...] = m1

    tt = lax.broadcasted_iota(jnp.int32, (NT, E), 0).astype(jnp.float32)
    ind = ((tt >= tile_start) & (tt < tile_start + tp)).astype(jnp.float32)
    ecol = lax.broadcasted_iota(jnp.int32, (NT, E), 1).astype(jnp.float32)
    te = jnp.sum(ind * ecol, axis=1, keepdims=True)
    tv = jnp.sum(ind, axis=1, keepdims=True)
    te_last = jnp.max(ind * ecol)
    te_ref[...] = jnp.where(tv > 0.0, te, te_last).astype(jnp.int32)
    tv_ref[...] = tv.astype(jnp.int32)


def _routing(xf, gate_W, gate_b, interpret=False):
    f32 = jnp.float32
    return pl.pallas_call(
        _routing_body,
        out_shape=(
            jax.ShapeDtypeStruct((T, 1), jnp.int32),
            jax.ShapeDtypeStruct((T, 1), jnp.int32),
            jax.ShapeDtypeStruct((T, 1), f32),
            jax.ShapeDtypeStruct((T, 1), f32),
            jax.ShapeDtypeStruct((NT, 1), jnp.int32),
            jax.ShapeDtypeStruct((NT, 1), jnp.int32),
        ),
        interpret=interpret,
    )(xf, gate_W, gate_b)


def _ffn_body(te_ref, tv_ref, x_ref, w1_ref, b1_ref, w2_ref, b2_ref, y_ref):
    t = pl.program_id(0)
    k = pl.program_id(1)

    @pl.when(tv_ref[t] > 0)
    def _():
        xb = x_ref[...].astype(jnp.bfloat16)
        h = jnp.dot(xb, w1_ref[0].astype(jnp.bfloat16),
                    preferred_element_type=jnp.float32)
        h = jnp.maximum(h + b1_ref[0], 0.0)
        part = jnp.dot(h.astype(jnp.bfloat16), w2_ref[0].astype(jnp.bfloat16),
                       preferred_element_type=jnp.float32)

        @pl.when(k == 0)
        def _():
            y_ref[...] = part + b2_ref[0]

        @pl.when(k > 0)
        def _():
            y_ref[...] = y_ref[...] + part


def _ffn(te, tv, xs, W1, b1, W2, b2, interpret=False):
    grid_spec = pltpu.PrefetchScalarGridSpec(
        num_scalar_prefetch=2,
        grid=(NT, NK),
        in_specs=[
            pl.BlockSpec((TM, D), lambda t, k, te, tv: (jnp.where(tv[t] > 0, t, 0), 0)),
            pl.BlockSpec((1, D, FFC),
                         lambda t, k, te, tv: (te[t], 0, jnp.where(tv[t] > 0, k, NK - 1))),
            pl.BlockSpec((1, 1, FFC),
                         lambda t, k, te, tv: (te[t] * NK + jnp.where(tv[t] > 0, k, NK - 1), 0, 0)),
            pl.BlockSpec((1, FFC, D),
                         lambda t, k, te, tv: (te[t], jnp.where(tv[t] > 0, k, NK - 1), 0)),
            pl.BlockSpec((1, 1, D), lambda t, k, te, tv: (te[t], 0, 0)),
        ],
        out_specs=pl.BlockSpec((TM, D),
                               lambda t, k, te, tv: (jnp.where(tv[t] > 0, t, NT - 1), 0)),
    )
    return pl.pallas_call(
        _ffn_body,
        grid_spec=grid_spec,
        out_shape=jax.ShapeDtypeStruct((P, D), jnp.float32),
        interpret=interpret,
    )(te, tv, xs, W1, b1.reshape(E * NK, 1, FFC), W2, b2.reshape(E, 1, D))


def _dispatch(xf, dest):
    mesh = plsc.VectorSubcoreMesh(core_axis_name="c", subcore_axis_name="s")

    @functools.partial(
        pl.kernel,
        out_type=jax.ShapeDtypeStruct((P, D), jnp.float32),
        mesh=mesh,
        scratch_types=[
            pltpu.VMEM((2, PAIRS_W // 2), jnp.int32),
            pltpu.VMEM((PAIRS_W, D), jnp.float32),
            pltpu.SemaphoreType.DMA,
            pltpu.SemaphoreType.DMA,
            pltpu.SemaphoreType.DMA,
        ],
    )
    def disp(xf_hbm, dest_hbm, xs_hbm, idx_v, rows_v, sem0, sem1, sem2):
        wid = lax.axis_index("s") * 2 + lax.axis_index("c")
        base = wid * PAIRS_W
        half = PAIRS_W // 2
        pltpu.sync_copy(dest_hbm.at[pl.ds(base, half)], idx_v.at[0])
        pltpu.sync_copy(dest_hbm.at[pl.ds(base + half, half)], idx_v.at[1])
        src = lax.rem(base, T)
        c0 = pltpu.async_copy(xf_hbm.at[pl.ds(src, half)],
                              rows_v.at[pl.ds(0, half)], sem0)
        c1 = pltpu.async_copy(xf_hbm.at[pl.ds(src + half, half)],
                              rows_v.at[pl.ds(half, half)], sem1)
        c0.wait()
        s0c = pltpu.async_copy(rows_v.at[pl.ds(0, half)],
                               xs_hbm.at[idx_v.at[0]], sem2)
        c1.wait()
        s1c = pltpu.async_copy(rows_v.at[pl.ds(half, half)],
                               xs_hbm.at[idx_v.at[1]], sem0)
        s0c.wait()
        s1c.wait()

    return disp(xf, dest)


def _combine(y, dest, sflat):
    mesh = plsc.VectorSubcoreMesh(core_axis_name="c", subcore_axis_name="s")

    @functools.partial(
        pl.kernel,
        out_type=jax.ShapeDtypeStruct((T, D), jnp.float32),
        mesh=mesh,
        scratch_types=[
            pltpu.VMEM((TOK_W,), jnp.int32),
            pltpu.VMEM((TOK_W,), jnp.int32),
            pltpu.VMEM((TOK_W + 16,), jnp.float32),
            pltpu.VMEM((TOK_W + 16,), jnp.float32),
            pltpu.VMEM((TOK_W, D), jnp.float32),
            pltpu.VMEM((TOK_W, D), jnp.float32),
            pltpu.SemaphoreType.DMA,
            pltpu.SemaphoreType.DMA,
        ],
    )
    def comb(y_hbm, dest_hbm, s_hbm, out_hbm,
             d0_v, d1_v, s0_v, s1_v, r0_v, r1_v, sem, semb):
        wid = lax.axis_index("s") * 2 + lax.axis_index("c")
        tb = wid * TOK_W
        pltpu.sync_copy(dest_hbm.at[pl.ds(tb, TOK_W)], d0_v)
        pltpu.sync_copy(dest_hbm.at[pl.ds(T + tb, TOK_W)], d1_v)
        pltpu.sync_copy(s_hbm.at[pl.ds(tb, TOK_W)], s0_v.at[pl.ds(0, TOK_W)])
        pltpu.sync_copy(s_hbm.at[pl.ds(T + tb, TOK_W)], s1_v.at[pl.ds(0, TOK_W)])
        half = TOK_W // 2
        g0a = pltpu.async_copy(y_hbm.at[d0_v.at[pl.ds(0, half)]],
                               r0_v.at[pl.ds(0, half)], sem)
        g0b = pltpu.async_copy(y_hbm.at[d1_v.at[pl.ds(0, half)]],
                               r1_v.at[pl.ds(0, half)], sem)
        g1a = pltpu.async_copy(y_hbm.at[d0_v.at[pl.ds(half, half)]],
                               r0_v.at[pl.ds(half, half)], semb)
        g1b = pltpu.async_copy(y_hbm.at[d1_v.at[pl.ds(half, half)]],
                               r1_v.at[pl.ds(half, half)], semb)

        def tok_body(tok, carry):
            s0b = jnp.full((16,), s0_v[pl.ds(tok, 16)][0], jnp.float32)
            s1b = jnp.full((16,), s1_v[pl.ds(tok, 16)][0], jnp.float32)
            for j in range(D // 16):
                sl = pl.ds(j * 16, 16)
                r0_v[tok, sl] = r0_v[tok, sl] * s0b + r1_v[tok, sl] * s1b
            return carry

        g0a.wait()
        g0b.wait()
        lax.fori_loop(0, half, tok_body, 0)
        w0 = pltpu.async_copy(r0_v.at[pl.ds(0, half)],
                              out_hbm.at[pl.ds(tb, half)], sem)
        g1a.wait()
        g1b.wait()
        lax.fori_loop(half, TOK_W, tok_body, 0)
        w1 = pltpu.async_copy(r0_v.at[pl.ds(half, half)],
                              out_hbm.at[pl.ds(tb + half, half)], semb)
        w0.wait()
        w1.wait()

    return comb(y, dest, sflat)


def kernel(x, gate_W, gate_b, W1, b1, W2, b2):
    xf = x.reshape(T, D)
    d0, d1, s0, s1, te, tv = _routing(xf, gate_W, gate_b.reshape(1, E))
    dest = jnp.concatenate([d0[:, 0], d1[:, 0]])
    sflat = jnp.concatenate([s0[:, 0], s1[:, 0]])
    xs = _dispatch(xf, dest)
    y = _ffn(te[:, 0], tv[:, 0], xs, W1, b1, W2, b2)
    out = _combine(y, dest, sflat)
    return out.reshape(1, T, D)
